# Initial kernel scaffold; baseline (speedup 1.0000x reference)
import jax
import jax.numpy as jnp
from jax.experimental import pallas as pl


def kernel(x, emb, W1, b1, W2, b2, W3, b3, W4, b4, W5, b5, W6, b6):
    raise NotImplementedError("write your pallas kernel here")


# speedup vs baseline: 1.9427x; 1.9427x over previous
import functools

import jax
import jax.numpy as jnp
from jax import lax
from jax.experimental import pallas as pl
from jax.experimental.pallas import tpu as pltpu
from jax.experimental.pallas import tpu_sc as plsc

_N_PAD = 1024
_B = 16384
_LANES = 16


def _mlp_table_body(emb_ref, w1_ref, b1_ref, w2_ref, b2_ref, w3_ref, b3_ref,
                    w4_ref, b4_ref, w5_ref, b5_ref, w6_ref, b6_ref, out_ref):
    e = emb_ref[...]
    w1 = w1_ref[...]
    h = e[:, 0:1] * w1[0:1, :] + e[:, 1:2] * w1[1:2, :] + b1_ref[...]
    h = jnp.maximum(h, 0.0)
    for w_ref, b_ref in ((w2_ref, b2_ref), (w3_ref, b3_ref),
                         (w4_ref, b4_ref), (w5_ref, b5_ref)):
        h = jnp.dot(h, w_ref[...], preferred_element_type=jnp.float32)
        h = jnp.maximum(h + b_ref[...], 0.0)
    logits = jnp.dot(h, w6_ref[...], preferred_element_type=jnp.float32)
    logits = logits + b6_ref[...]
    out_ref[...] = jax.nn.sigmoid(logits)


def _make_gather_kernel():
    info = plsc.get_sparse_core_info()
    nw = info.num_cores * info.num_subcores
    b_per_w = _B // nw
    mesh = plsc.VectorSubcoreMesh(core_axis_name="c", subcore_axis_name="s")

    @functools.partial(
        pl.kernel,
        mesh=mesh,
        compiler_params=pltpu.CompilerParams(needs_layout_passes=False),
        out_type=jax.ShapeDtypeStruct((_B,), jnp.float32),
        scratch_types=[
            pltpu.VMEM((b_per_w,), jnp.int32),
            pltpu.VMEM((_N_PAD,), jnp.float32),
            pltpu.VMEM((b_per_w,), jnp.float32),
        ],
    )
    def gather_k(table_hbm, idx_hbm, out_hbm, idx_v, table_v, out_v):
        wid = lax.axis_index("s") * info.num_cores + lax.axis_index("c")
        base = wid * b_per_w
        pltpu.sync_copy(idx_hbm.at[pl.ds(base, b_per_w)], idx_v)
        pltpu.sync_copy(table_hbm, table_v)
        for i in range(b_per_w // _LANES):
            idx16 = idx_v[pl.ds(i * _LANES, _LANES)]
            out_v[pl.ds(i * _LANES, _LANES)] = plsc.load_gather(table_v, [idx16])
        pltpu.sync_copy(out_v, out_hbm.at[pl.ds(base, b_per_w)])

    return gather_k


_gather_kernel = None


def kernel(x, emb, W1, b1, W2, b2, W3, b3, W4, b4, W5, b5, W6, b6):
    global _gather_kernel
    n_users = emb.shape[0]
    emb_pad = jnp.zeros((_N_PAD, emb.shape[1]), emb.dtype).at[:n_users].set(emb)

    table = pl.pallas_call(
        _mlp_table_body,
        out_shape=jax.ShapeDtypeStruct((_N_PAD, 1), jnp.float32),
    )(emb_pad, W1, b1.reshape(1, -1), W2, b2.reshape(1, -1),
      W3, b3.reshape(1, -1), W4, b4.reshape(1, -1), W5, b5.reshape(1, -1),
      W6, b6.reshape(1, -1))
    table = table.reshape(-1)

    if _gather_kernel is None:
        _gather_kernel = _make_gather_kernel()
    return _gather_kernel(table, x.astype(jnp.int32))


# speedup vs baseline: 2.1033x; 1.0826x over previous
import functools

import jax
import jax.numpy as jnp
from jax import lax
from jax.experimental import pallas as pl
from jax.experimental.pallas import tpu as pltpu
from jax.experimental.pallas import tpu_sc as plsc

_N_PAD = 1024
_B = 16384
_LANES = 16


def _mlp_table_body(emb_ref, w1_ref, b1_ref, w2_ref, b2_ref, w3_ref, b3_ref,
                    w4_ref, b4_ref, w5_ref, b5_ref, w6_ref, b6_ref, out_ref):
    n = emb_ref.shape[0]
    et = emb_ref[...].T
    w1 = w1_ref[...]
    h = (et[0:1, :] * w1[0:1, :].T + et[1:2, :] * w1[1:2, :].T
         + b1_ref[...].reshape(-1, 1))
    h = jnp.maximum(h, 0.0)
    for w_ref, b_ref in ((w2_ref, b2_ref), (w3_ref, b3_ref),
                         (w4_ref, b4_ref), (w5_ref, b5_ref)):
        h = lax.dot_general(w_ref[...], h, (((0,), (0,)), ((), ())),
                            preferred_element_type=jnp.float32)
        h = jnp.maximum(h + b_ref[...].reshape(-1, 1), 0.0)
    logits = lax.dot_general(w6_ref[...], h, (((0,), (0,)), ((), ())),
                             preferred_element_type=jnp.float32)
    sig = jax.nn.sigmoid(logits[0, :] + b6_ref[0])
    out_ref[...] = jnp.concatenate([sig, jnp.zeros((_N_PAD - n,), jnp.float32)])


def _make_gather_kernel():
    info = plsc.get_sparse_core_info()
    nw = info.num_cores * info.num_subcores
    b_per_w = _B // nw
    mesh = plsc.VectorSubcoreMesh(core_axis_name="c", subcore_axis_name="s")

    @functools.partial(
        pl.kernel,
        mesh=mesh,
        compiler_params=pltpu.CompilerParams(needs_layout_passes=False),
        out_type=jax.ShapeDtypeStruct((_B,), jnp.float32),
        scratch_types=[
            pltpu.VMEM((b_per_w,), jnp.int32),
            pltpu.VMEM((_N_PAD,), jnp.float32),
            pltpu.VMEM((b_per_w,), jnp.float32),
        ],
    )
    def gather_k(table_hbm, idx_hbm, out_hbm, idx_v, table_v, out_v):
        wid = lax.axis_index("s") * info.num_cores + lax.axis_index("c")
        base = wid * b_per_w
        pltpu.sync_copy(idx_hbm.at[pl.ds(base, b_per_w)], idx_v)
        pltpu.sync_copy(table_hbm, table_v)
        for i in range(b_per_w // _LANES):
            idx16 = idx_v[pl.ds(i * _LANES, _LANES)]
            out_v[pl.ds(i * _LANES, _LANES)] = plsc.load_gather(table_v, [idx16])
        pltpu.sync_copy(out_v, out_hbm.at[pl.ds(base, b_per_w)])

    return gather_k


_gather_kernel = None


def kernel(x, emb, W1, b1, W2, b2, W3, b3, W4, b4, W5, b5, W6, b6):
    global _gather_kernel
    table = pl.pallas_call(
        _mlp_table_body,
        out_shape=jax.ShapeDtypeStruct((_N_PAD,), jnp.float32),
    )(emb, W1, b1, W2, b2, W3, b3, W4, b4, W5, b5, W6, b6)

    if _gather_kernel is None:
        _gather_kernel = _make_gather_kernel()
    return _gather_kernel(table, x.astype(jnp.int32))
